# Initial kernel scaffold; baseline (speedup 1.0000x reference)
import jax
import jax.numpy as jnp
from jax.experimental import pallas as pl


def kernel(output_tokens, output_scores, cur_tokens, cur_scores, xt_neq_x0, non_special_sym_mask, t, max_step, noise):
    raise NotImplementedError("write your pallas kernel here")


# speedup vs baseline: 5.4273x; 5.4273x over previous
import jax
import jax.numpy as jnp
from jax.experimental import pallas as pl
from jax.experimental.pallas import tpu as pltpu

_ROWS_PER_BLOCK = 8


def _body(rate_ref, noise_ref, ot_ref, os_ref, ct_ref, cs_ref, xt_ref, mask_ref,
          tok_out, sc_out, nxt_out):
    rate = rate_ref[0]
    noise = noise_ref[0]
    mask = mask_ref[...]
    cs = cs_ref[...]
    sft = jnp.where(mask, cs, jnp.float32(1000.0))
    keys = jax.lax.bitcast_convert_type(sft, jnp.int32)
    cnt_row = jnp.sum(mask.astype(jnp.float32), axis=1, keepdims=True)
    target = (cnt_row * rate).astype(jnp.int32) + 1
    ans = jnp.zeros_like(target)
    for bit in range(30, -1, -1):
        mid = ans | ((1 << bit) - 1)
        cnt = jnp.sum((keys <= mid).astype(jnp.int32), axis=1, keepdims=True)
        ans = jnp.where(cnt >= target, ans, ans + (1 << bit))
    m = keys < ans
    xt = xt_ref[...]
    tok_out[...] = jnp.where(m, noise, jnp.where(xt, ct_ref[...], ot_ref[...]))
    sc_out[...] = jnp.where(m, jnp.float32(-jnp.inf),
                            jnp.where(xt, cs, os_ref[...]))
    nxt_out[...] = m


def kernel(output_tokens, output_scores, cur_tokens, cur_scores, xt_neq_x0,
           non_special_sym_mask, t, max_step, noise):
    B, N = cur_scores.shape
    R = _ROWS_PER_BLOCK
    rate = (1.0 - (t + 1) / max_step).astype(jnp.float32).reshape(1)
    noise_arr = jnp.asarray(noise, jnp.int32).reshape(1)

    row_block = pl.BlockSpec((R, N), lambda i: (i, 0))
    smem_spec = pl.BlockSpec(memory_space=pltpu.SMEM)
    out_shapes = (
        jax.ShapeDtypeStruct((B, N), output_tokens.dtype),
        jax.ShapeDtypeStruct((B, N), output_scores.dtype),
        jax.ShapeDtypeStruct((B, N), jnp.bool_),
    )
    return pl.pallas_call(
        _body,
        grid=(B // R,),
        in_specs=[smem_spec, smem_spec] + [row_block] * 6,
        out_specs=(row_block, row_block, row_block),
        out_shape=out_shapes,
    )(rate, noise_arr, output_tokens, output_scores, cur_tokens, cur_scores,
      xt_neq_x0, non_special_sym_mask)


# speedup vs baseline: 8.8701x; 1.6344x over previous
import jax
import jax.numpy as jnp
from jax.experimental import pallas as pl
from jax.experimental.pallas import tpu as pltpu

_ROWS_PER_BLOCK = 32


def _body(rate_ref, noise_ref, ot_ref, os_ref, ct_ref, cs_ref, xt_ref, mask_ref,
          tok_out, sc_out, nxt_out):
    rate = rate_ref[0]
    noise = noise_ref[0]
    mask = mask_ref[...]
    cs = cs_ref[...]
    sft = jnp.where(mask, cs, jnp.float32(1000.0))
    keys = jax.lax.bitcast_convert_type(sft, jnp.int32)
    cnt_row = jnp.sum(mask.astype(jnp.float32), axis=1, keepdims=True)
    target = (cnt_row * rate).astype(jnp.int32) + 1
    ans = jnp.zeros_like(target)
    for bit in range(30, -1, -1):
        mid = ans | ((1 << bit) - 1)
        cnt = jnp.sum((keys <= mid).astype(jnp.int32), axis=1, keepdims=True)
        ans = jnp.where(cnt >= target, ans, ans + (1 << bit))
    m = keys < ans
    xt = xt_ref[...]
    tok_out[...] = jnp.where(m, noise, jnp.where(xt, ct_ref[...], ot_ref[...]))
    sc_out[...] = jnp.where(m, jnp.float32(-jnp.inf),
                            jnp.where(xt, cs, os_ref[...]))
    nxt_out[...] = m


def kernel(output_tokens, output_scores, cur_tokens, cur_scores, xt_neq_x0,
           non_special_sym_mask, t, max_step, noise):
    B, N = cur_scores.shape
    R = _ROWS_PER_BLOCK
    rate = (1.0 - (t + 1) / max_step).astype(jnp.float32).reshape(1)
    noise_arr = jnp.asarray(noise, jnp.int32).reshape(1)

    row_block = pl.BlockSpec((R, N), lambda i: (i, 0))
    smem_spec = pl.BlockSpec(memory_space=pltpu.SMEM)
    out_shapes = (
        jax.ShapeDtypeStruct((B, N), output_tokens.dtype),
        jax.ShapeDtypeStruct((B, N), output_scores.dtype),
        jax.ShapeDtypeStruct((B, N), jnp.bool_),
    )
    return pl.pallas_call(
        _body,
        grid=(B // R,),
        in_specs=[smem_spec, smem_spec] + [row_block] * 6,
        out_specs=(row_block, row_block, row_block),
        out_shape=out_shapes,
    )(rate, noise_arr, output_tokens, output_scores, cur_tokens, cur_scores,
      xt_neq_x0, non_special_sym_mask)


# speedup vs baseline: 9.4403x; 1.0643x over previous
import jax
import jax.numpy as jnp
from jax.experimental import pallas as pl
from jax.experimental.pallas import tpu as pltpu

_ROWS_PER_BLOCK = 64


def _body(rate_ref, noise_ref, ot_ref, os_ref, ct_ref, cs_ref, xt_ref, mask_ref,
          tok_out, sc_out, nxt_out):
    rate = rate_ref[0]
    noise = noise_ref[0]
    mask = mask_ref[...]
    cs = cs_ref[...]
    sft = jnp.where(mask, cs, jnp.float32(1000.0))
    keys = jax.lax.bitcast_convert_type(sft, jnp.int32)
    cnt_row = jnp.sum(mask.astype(jnp.float32), axis=1, keepdims=True)
    target = (cnt_row * rate).astype(jnp.int32) + 1
    ans = jnp.zeros_like(target)
    for bit in range(30, -1, -1):
        mid = ans | ((1 << bit) - 1)
        cnt = jnp.sum((keys <= mid).astype(jnp.int32), axis=1, keepdims=True)
        ans = jnp.where(cnt >= target, ans, ans + (1 << bit))
    m = keys < ans
    xt = xt_ref[...]
    tok_out[...] = jnp.where(m, noise, jnp.where(xt, ct_ref[...], ot_ref[...]))
    sc_out[...] = jnp.where(m, jnp.float32(-jnp.inf),
                            jnp.where(xt, cs, os_ref[...]))
    nxt_out[...] = m


def kernel(output_tokens, output_scores, cur_tokens, cur_scores, xt_neq_x0,
           non_special_sym_mask, t, max_step, noise):
    B, N = cur_scores.shape
    R = _ROWS_PER_BLOCK
    rate = (1.0 - (t + 1) / max_step).astype(jnp.float32).reshape(1)
    noise_arr = jnp.asarray(noise, jnp.int32).reshape(1)

    row_block = pl.BlockSpec((R, N), lambda i: (i, 0))
    smem_spec = pl.BlockSpec(memory_space=pltpu.SMEM)
    out_shapes = (
        jax.ShapeDtypeStruct((B, N), output_tokens.dtype),
        jax.ShapeDtypeStruct((B, N), output_scores.dtype),
        jax.ShapeDtypeStruct((B, N), jnp.bool_),
    )
    return pl.pallas_call(
        _body,
        grid=(B // R,),
        in_specs=[smem_spec, smem_spec] + [row_block] * 6,
        out_specs=(row_block, row_block, row_block),
        out_shape=out_shapes,
    )(rate, noise_arr, output_tokens, output_scores, cur_tokens, cur_scores,
      xt_neq_x0, non_special_sym_mask)


# speedup vs baseline: 10.9770x; 1.1628x over previous
import jax
import jax.numpy as jnp
from jax.experimental import pallas as pl
from jax.experimental.pallas import tpu as pltpu

_ROWS_PER_BLOCK = 64


def _count16(hits):
    w = hits.shape[1]
    while w > 128:
        w //= 2
        hits = hits[:, :w] + hits[:, w:]
    return jnp.sum(hits.astype(jnp.int32), axis=1, keepdims=True)


def _body(rate_ref, noise_ref, ot_ref, os_ref, ct_ref, cs_ref, xt_ref, mask_ref,
          tok_out, sc_out, nxt_out):
    rate = rate_ref[0]
    noise = noise_ref[0]
    mask = mask_ref[...]
    cs = cs_ref[...]
    sft = jnp.where(mask, cs, jnp.float32(1000.0))
    keys = jax.lax.bitcast_convert_type(sft, jnp.int32)
    hi16 = (keys >> 16).astype(jnp.int16)
    lo16b = keys.astype(jnp.int16) ^ jnp.int16(-32768)
    cnt_row = jnp.sum(mask.astype(jnp.float32), axis=1, keepdims=True)
    target = (cnt_row * rate).astype(jnp.int32) + 1

    one16 = jnp.int16(1)
    zero16 = jnp.int16(0)
    ansh = jnp.zeros_like(target)
    for bit in range(14, -1, -1):
        mid = ansh | ((1 << bit) - 1)
        hits = jnp.where(hi16 <= mid.astype(jnp.int16), one16, zero16)
        cnt = _count16(hits)
        ansh = jnp.where(cnt >= target, ansh, ansh + (1 << bit))

    anshi16 = ansh.astype(jnp.int16)
    eq = hi16 == anshi16
    base = _count16(jnp.where(hi16 < anshi16, one16, zero16))
    lok = jnp.where(eq, lo16b, jnp.int16(32767))
    targ2 = target - base

    ansl = jnp.zeros_like(target)
    for bit in range(15, -1, -1):
        mid = ansl | ((1 << bit) - 1)
        hits = jnp.where(lok <= (mid ^ 32768).astype(jnp.int16), one16, zero16)
        cnt = _count16(hits)
        ansl = jnp.where(cnt >= targ2, ansl, ansl + (1 << bit))

    ans = (ansh << 16) | ansl
    m = keys < ans
    xt = xt_ref[...]
    tok_out[...] = jnp.where(m, noise, jnp.where(xt, ct_ref[...], ot_ref[...]))
    sc_out[...] = jnp.where(m, jnp.float32(-jnp.inf),
                            jnp.where(xt, cs, os_ref[...]))
    nxt_out[...] = m


def kernel(output_tokens, output_scores, cur_tokens, cur_scores, xt_neq_x0,
           non_special_sym_mask, t, max_step, noise):
    B, N = cur_scores.shape
    R = _ROWS_PER_BLOCK
    rate = (1.0 - (t + 1) / max_step).astype(jnp.float32).reshape(1)
    noise_arr = jnp.asarray(noise, jnp.int32).reshape(1)

    row_block = pl.BlockSpec((R, N), lambda i: (i, 0))
    smem_spec = pl.BlockSpec(memory_space=pltpu.SMEM)
    out_shapes = (
        jax.ShapeDtypeStruct((B, N), output_tokens.dtype),
        jax.ShapeDtypeStruct((B, N), output_scores.dtype),
        jax.ShapeDtypeStruct((B, N), jnp.bool_),
    )
    return pl.pallas_call(
        _body,
        grid=(B // R,),
        in_specs=[smem_spec, smem_spec] + [row_block] * 6,
        out_specs=(row_block, row_block, row_block),
        out_shape=out_shapes,
    )(rate, noise_arr, output_tokens, output_scores, cur_tokens, cur_scores,
      xt_neq_x0, non_special_sym_mask)
